# baseline (speedup 1.0000x reference)
import math

import jax
import jax.numpy as jnp
from jax.experimental import pallas as pl
from jax.experimental.pallas import tpu as pltpu

_TILE = 2048
_NEG = -1e30


def _attn_kernel(starts_ref, ends_ref, enc_ref, wqt_ref, bq_ref, wkt_ref,
                 bk_ref, wvt_ref, bv_ref, social_ref, out_ref,
                 q_ref, m_ref, s_ref, acc_ref):
    j = pl.program_id(0)
    nt = pl.num_programs(0)
    b, d = out_ref.shape
    tile = social_ref.shape[0]

    @pl.when(j == 0)
    def _init():
        q = jnp.dot(enc_ref[...].astype(jnp.bfloat16), wqt_ref[...],
                    preferred_element_type=jnp.float32) + bq_ref[...]
        q_ref[...] = jnp.maximum(q, 0.0) * (1.0 / math.sqrt(d))
        m_ref[...] = jnp.full((b, d), _NEG, jnp.float32)
        s_ref[...] = jnp.zeros((b, d), jnp.float32)
        acc_ref[...] = jnp.zeros((b, d), jnp.float32)

    max_end = jnp.max(ends_ref[...])

    @pl.when(j * tile < max_end)
    def _tile():
        tok = social_ref[...].astype(jnp.bfloat16)
        k = jnp.maximum(jnp.dot(tok, wkt_ref[...],
                                preferred_element_type=jnp.float32) + bk_ref[...], 0.0)
        v = jnp.maximum(jnp.dot(tok, wvt_ref[...],
                                preferred_element_type=jnp.float32) + bv_ref[...], 0.0)

        logits = jax.lax.dot_general(
            q_ref[...].astype(jnp.bfloat16), k.astype(jnp.bfloat16),
            (((1,), (1,)), ((), ())),
            preferred_element_type=jnp.float32)
        col = j * tile + jax.lax.broadcasted_iota(jnp.int32, (b, tile), 1)
        mask = (col >= starts_ref[...]) & (col < ends_ref[...])
        logits = jnp.where(mask, logits, _NEG)

        m_old = m_ref[...]
        row_max = jnp.max(logits, axis=1, keepdims=True)
        m_new = jnp.maximum(m_old, row_max)
        alpha = jnp.exp(m_old - m_new)
        p = jnp.exp(logits - jnp.max(m_new, axis=1, keepdims=True))

        s_ref[...] = s_ref[...] * alpha + jnp.sum(p, axis=1, keepdims=True)
        acc_ref[...] = acc_ref[...] * alpha + jnp.dot(
            p.astype(jnp.bfloat16), v.astype(jnp.bfloat16),
            preferred_element_type=jnp.float32)
        m_ref[...] = m_new

    @pl.when(j == nt - 1)
    def _fin():
        out_ref[...] = acc_ref[...] / s_ref[...]


def kernel(enc_hidden, social_ht, neighbors_idx_start, neighbors_idx_end,
           Wq, bq, Wk, bk, Wv, bv):
    b, d = enc_hidden.shape
    t = social_ht.shape[0]
    nt = t // _TILE

    starts = neighbors_idx_start.astype(jnp.int32).reshape(b, 1)
    ends = neighbors_idx_end.astype(jnp.int32).reshape(b, 1)

    const = lambda j: (0, 0)
    out = pl.pallas_call(
        _attn_kernel,
        grid=(nt,),
        in_specs=[
            pl.BlockSpec((b, 1), const),
            pl.BlockSpec((b, 1), const),
            pl.BlockSpec((b, d), const),
            pl.BlockSpec((d, d), const),
            pl.BlockSpec((1, d), const),
            pl.BlockSpec((d, d), const),
            pl.BlockSpec((1, d), const),
            pl.BlockSpec((d, d), const),
            pl.BlockSpec((1, d), const),
            pl.BlockSpec((_TILE, d), lambda j: (j, 0)),
        ],
        out_specs=pl.BlockSpec((b, d), const),
        out_shape=jax.ShapeDtypeStruct((b, d), jnp.float32),
        scratch_shapes=[
            pltpu.VMEM((b, d), jnp.float32),
            pltpu.VMEM((b, d), jnp.float32),
            pltpu.VMEM((b, d), jnp.float32),
            pltpu.VMEM((b, d), jnp.float32),
        ],
        compiler_params=pltpu.CompilerParams(
            dimension_semantics=("arbitrary",)),
    )(starts, ends, enc_hidden,
      Wq.T.astype(jnp.bfloat16), bq.reshape(1, d),
      Wk.T.astype(jnp.bfloat16), bk.reshape(1, d),
      Wv.T.astype(jnp.bfloat16), bv.reshape(1, d), social_ht)
    return out


# speedup vs baseline: 1.3470x; 1.3470x over previous
import math

import jax
import jax.numpy as jnp
from jax.experimental import pallas as pl
from jax.experimental.pallas import tpu as pltpu

_TILE = 8192
_NEG = -1e30


def _attn_kernel(starts_ref, ends_ref, enc_ref, wqt_ref, bq_ref, wkt_ref,
                 bk_ref, wvt_ref, bv_ref, social_ref, out_ref,
                 q_ref, m_ref, s_ref, acc_ref):
    j = pl.program_id(0)
    nt = pl.num_programs(0)
    b, d = out_ref.shape
    tile = social_ref.shape[0]

    @pl.when(j == 0)
    def _init():
        q = jnp.dot(enc_ref[...], wqt_ref[...],
                    preferred_element_type=jnp.float32) + bq_ref[...]
        q_ref[...] = jnp.maximum(q, 0.0) * (1.0 / math.sqrt(d))
        m_ref[...] = jnp.full((b, d), _NEG, jnp.float32)
        s_ref[...] = jnp.zeros((b, d), jnp.float32)
        acc_ref[...] = jnp.zeros((b, d), jnp.float32)

    max_end = jnp.max(ends_ref[...])

    @pl.when(j * tile < max_end)
    def _tile():
        tok = social_ref[...]
        k = jnp.maximum(jnp.dot(tok, wkt_ref[...],
                                preferred_element_type=jnp.float32) + bk_ref[...], 0.0)
        v = jnp.maximum(jnp.dot(tok, wvt_ref[...],
                                preferred_element_type=jnp.float32) + bv_ref[...], 0.0)

        logits = jax.lax.dot_general(
            q_ref[...], k, (((1,), (1,)), ((), ())),
            preferred_element_type=jnp.float32)
        col = j * tile + jax.lax.broadcasted_iota(jnp.int32, (b, tile), 1)
        mask = (col >= starts_ref[...]) & (col < ends_ref[...])
        logits = jnp.where(mask, logits, _NEG)

        m_old = m_ref[...]
        row_max = jnp.max(logits, axis=1, keepdims=True)
        m_new = jnp.maximum(m_old, row_max)
        alpha = jnp.exp(m_old - m_new)
        p = jnp.exp(logits - jnp.max(m_new, axis=1, keepdims=True))

        s_ref[...] = s_ref[...] * alpha + jnp.sum(p, axis=1, keepdims=True)
        acc_ref[...] = acc_ref[...] * alpha + jnp.dot(
            p, v, preferred_element_type=jnp.float32)
        m_ref[...] = m_new

    @pl.when(j == nt - 1)
    def _fin():
        out_ref[...] = acc_ref[...] / s_ref[...]


def kernel(enc_hidden, social_ht, neighbors_idx_start, neighbors_idx_end,
           Wq, bq, Wk, bk, Wv, bv):
    b, d = enc_hidden.shape
    t = social_ht.shape[0]
    nt = t // _TILE

    starts = neighbors_idx_start.astype(jnp.int32).reshape(b, 1)
    ends = neighbors_idx_end.astype(jnp.int32).reshape(b, 1)

    const = lambda j: (0, 0)
    out = pl.pallas_call(
        _attn_kernel,
        grid=(nt,),
        in_specs=[
            pl.BlockSpec((b, 1), const),
            pl.BlockSpec((b, 1), const),
            pl.BlockSpec((b, d), const),
            pl.BlockSpec((d, d), const),
            pl.BlockSpec((1, d), const),
            pl.BlockSpec((d, d), const),
            pl.BlockSpec((1, d), const),
            pl.BlockSpec((d, d), const),
            pl.BlockSpec((1, d), const),
            pl.BlockSpec((_TILE, d), lambda j: (j, 0)),
        ],
        out_specs=pl.BlockSpec((b, d), const),
        out_shape=jax.ShapeDtypeStruct((b, d), jnp.float32),
        scratch_shapes=[
            pltpu.VMEM((b, d), jnp.float32),
            pltpu.VMEM((b, d), jnp.float32),
            pltpu.VMEM((b, d), jnp.float32),
            pltpu.VMEM((b, d), jnp.float32),
        ],
        compiler_params=pltpu.CompilerParams(
            dimension_semantics=("arbitrary",)),
    )(starts, ends, enc_hidden,
      Wq.T, bq.reshape(1, d),
      Wk.T, bk.reshape(1, d),
      Wv.T, bv.reshape(1, d), social_ht)
    return out


# speedup vs baseline: 1.3556x; 1.0064x over previous
import math

import jax
import jax.numpy as jnp
from jax.experimental import pallas as pl
from jax.experimental.pallas import tpu as pltpu

_TILE = 16384
_NEG = -1e30


def _attn_kernel(starts_ref, ends_ref, enc_ref, wqt_ref, bq_ref, wkt_ref,
                 bk_ref, wvt_ref, bv_ref, social_ref, out_ref,
                 q_ref, m_ref, s_ref, acc_ref):
    j = pl.program_id(0)
    nt = pl.num_programs(0)
    b, d = out_ref.shape
    tile = social_ref.shape[0]

    @pl.when(j == 0)
    def _init():
        q = jnp.dot(enc_ref[...], wqt_ref[...],
                    preferred_element_type=jnp.float32) + bq_ref[...]
        q_ref[...] = jnp.maximum(q, 0.0) * (1.0 / math.sqrt(d))
        m_ref[...] = jnp.full((b, d), _NEG, jnp.float32)
        s_ref[...] = jnp.zeros((b, d), jnp.float32)
        acc_ref[...] = jnp.zeros((b, d), jnp.float32)

    max_end = jnp.max(ends_ref[...])

    @pl.when(j * tile < max_end)
    def _tile():
        tok = social_ref[...]
        k = jnp.maximum(jnp.dot(tok, wkt_ref[...],
                                preferred_element_type=jnp.float32) + bk_ref[...], 0.0)
        v = jnp.maximum(jnp.dot(tok, wvt_ref[...],
                                preferred_element_type=jnp.float32) + bv_ref[...], 0.0)

        logits = jax.lax.dot_general(
            q_ref[...], k, (((1,), (1,)), ((), ())),
            preferred_element_type=jnp.float32)
        col = j * tile + jax.lax.broadcasted_iota(jnp.int32, (b, tile), 1)
        mask = (col >= starts_ref[...]) & (col < ends_ref[...])
        logits = jnp.where(mask, logits, _NEG)

        m_old = m_ref[...]
        row_max = jnp.max(logits, axis=1, keepdims=True)
        m_new = jnp.maximum(m_old, row_max)
        alpha = jnp.exp(m_old - m_new)
        p = jnp.exp(logits - jnp.max(m_new, axis=1, keepdims=True))

        s_ref[...] = s_ref[...] * alpha + jnp.sum(p, axis=1, keepdims=True)
        acc_ref[...] = acc_ref[...] * alpha + jnp.dot(
            p, v, preferred_element_type=jnp.float32)
        m_ref[...] = m_new

    @pl.when(j == nt - 1)
    def _fin():
        out_ref[...] = acc_ref[...] / s_ref[...]


def kernel(enc_hidden, social_ht, neighbors_idx_start, neighbors_idx_end,
           Wq, bq, Wk, bk, Wv, bv):
    b, d = enc_hidden.shape
    t = social_ht.shape[0]
    nt = t // _TILE

    starts = neighbors_idx_start.astype(jnp.int32).reshape(b, 1)
    ends = neighbors_idx_end.astype(jnp.int32).reshape(b, 1)

    const = lambda j: (0, 0)
    out = pl.pallas_call(
        _attn_kernel,
        grid=(nt,),
        in_specs=[
            pl.BlockSpec((b, 1), const),
            pl.BlockSpec((b, 1), const),
            pl.BlockSpec((b, d), const),
            pl.BlockSpec((d, d), const),
            pl.BlockSpec((1, d), const),
            pl.BlockSpec((d, d), const),
            pl.BlockSpec((1, d), const),
            pl.BlockSpec((d, d), const),
            pl.BlockSpec((1, d), const),
            pl.BlockSpec((_TILE, d), lambda j: (j, 0)),
        ],
        out_specs=pl.BlockSpec((b, d), const),
        out_shape=jax.ShapeDtypeStruct((b, d), jnp.float32),
        scratch_shapes=[
            pltpu.VMEM((b, d), jnp.float32),
            pltpu.VMEM((b, d), jnp.float32),
            pltpu.VMEM((b, d), jnp.float32),
            pltpu.VMEM((b, d), jnp.float32),
        ],
        compiler_params=pltpu.CompilerParams(
            dimension_semantics=("arbitrary",)),
    )(starts, ends, enc_hidden,
      Wq.T, bq.reshape(1, d),
      Wk.T, bk.reshape(1, d),
      Wv.T, bv.reshape(1, d), social_ht)
    return out
